# Initial kernel scaffold; baseline (speedup 1.0000x reference)
import jax
import jax.numpy as jnp
from jax.experimental import pallas as pl


def kernel(x, token_ids, gate_proj_w, up_proj_w, down_proj_w, shared_gate_w, shared_up_w, shared_down_w):
    raise NotImplementedError("write your pallas kernel here")


# speedup vs baseline: 4.0545x; 4.0545x over previous
import jax
import jax.numpy as jnp
from jax.experimental import pallas as pl

NUM_EXPERTS = 8
N_EMBD = 1024
EXPERT_DIM = 128
BN = 512


def _fused_body(tid_ref, x_ref, wsg_ref, wsu_ref, wsd_ref, wga_ref, wua_ref,
                wda_ref, out_ref):
    x = x_ref[...]
    g = jnp.dot(x, wsg_ref[...], preferred_element_type=jnp.float32)
    u = jnp.dot(x, wsu_ref[...], preferred_element_type=jnp.float32)
    s = g * jax.nn.sigmoid(g) * u
    sh = jnp.dot(s, wsd_ref[...], preferred_element_type=jnp.float32)
    eids = jax.lax.rem(tid_ref[0, 0, :], NUM_EXPERTS).reshape(BN, 1)
    col_expert = jax.lax.broadcasted_iota(jnp.int32, (BN, NUM_EXPERTS * EXPERT_DIM), 1) // EXPERT_DIM
    mask = (col_expert == eids).astype(jnp.float32)
    gr = jnp.dot(x, wga_ref[...], preferred_element_type=jnp.float32)
    ur = jnp.dot(x, wua_ref[...], preferred_element_type=jnp.float32)
    inter = gr * jax.nn.sigmoid(gr) * ur * mask
    r = jnp.dot(inter, wda_ref[...], preferred_element_type=jnp.float32)
    out_ref[...] = sh + r


def kernel(x, token_ids, gate_proj_w, up_proj_w, down_proj_w, shared_gate_w,
           shared_up_w, shared_down_w):
    b, t, h = x.shape
    n = b * t
    nb = n // BN
    flat_x = x.reshape(n, h)
    tid = token_ids.reshape(nb, 1, BN).astype(jnp.int32)
    wga = jnp.transpose(gate_proj_w, (1, 0, 2)).reshape(h, NUM_EXPERTS * EXPERT_DIM)
    wua = jnp.transpose(up_proj_w, (1, 0, 2)).reshape(h, NUM_EXPERTS * EXPERT_DIM)
    wda = down_proj_w.reshape(NUM_EXPERTS * EXPERT_DIM, h)

    out = pl.pallas_call(
        _fused_body,
        grid=(nb,),
        in_specs=[
            pl.BlockSpec((1, 1, BN), lambda i: (i, 0, 0)),
            pl.BlockSpec((BN, h), lambda i: (i, 0)),
            pl.BlockSpec((h, h), lambda i: (0, 0)),
            pl.BlockSpec((h, h), lambda i: (0, 0)),
            pl.BlockSpec((h, h), lambda i: (0, 0)),
            pl.BlockSpec((h, h), lambda i: (0, 0)),
            pl.BlockSpec((h, h), lambda i: (0, 0)),
            pl.BlockSpec((h, h), lambda i: (0, 0)),
        ],
        out_specs=pl.BlockSpec((BN, h), lambda i: (i, 0)),
        out_shape=jax.ShapeDtypeStruct((n, h), jnp.float32),
    )(tid, flat_x, shared_gate_w, shared_up_w, shared_down_w, wga, wua, wda)
    return out.reshape(b, t, h)


# speedup vs baseline: 4.0680x; 1.0033x over previous
import jax
import jax.numpy as jnp
from jax.experimental import pallas as pl

NUM_EXPERTS = 8
N_EMBD = 1024
EXPERT_DIM = 128
BN = 1024


def _fused_body(tid_ref, x_ref, wsg_ref, wsu_ref, wsd_ref, wga_ref, wua_ref,
                wda_ref, out_ref):
    x = x_ref[...]
    g = jnp.dot(x, wsg_ref[...], preferred_element_type=jnp.float32)
    u = jnp.dot(x, wsu_ref[...], preferred_element_type=jnp.float32)
    s = g * jax.nn.sigmoid(g) * u
    sh = jnp.dot(s, wsd_ref[...], preferred_element_type=jnp.float32)
    eids = jax.lax.rem(tid_ref[0, 0, :], NUM_EXPERTS).reshape(BN, 1)
    col_expert = jax.lax.broadcasted_iota(jnp.int32, (BN, NUM_EXPERTS * EXPERT_DIM), 1) // EXPERT_DIM
    mask = (col_expert == eids).astype(jnp.float32)
    gr = jnp.dot(x, wga_ref[...], preferred_element_type=jnp.float32)
    ur = jnp.dot(x, wua_ref[...], preferred_element_type=jnp.float32)
    inter = gr * jax.nn.sigmoid(gr) * ur * mask
    r = jnp.dot(inter, wda_ref[...], preferred_element_type=jnp.float32)
    out_ref[...] = sh + r


def kernel(x, token_ids, gate_proj_w, up_proj_w, down_proj_w, shared_gate_w,
           shared_up_w, shared_down_w):
    b, t, h = x.shape
    n = b * t
    nb = n // BN
    flat_x = x.reshape(n, h)
    tid = token_ids.reshape(nb, 1, BN).astype(jnp.int32)
    wga = jnp.transpose(gate_proj_w, (1, 0, 2)).reshape(h, NUM_EXPERTS * EXPERT_DIM)
    wua = jnp.transpose(up_proj_w, (1, 0, 2)).reshape(h, NUM_EXPERTS * EXPERT_DIM)
    wda = down_proj_w.reshape(NUM_EXPERTS * EXPERT_DIM, h)

    out = pl.pallas_call(
        _fused_body,
        grid=(nb,),
        in_specs=[
            pl.BlockSpec((1, 1, BN), lambda i: (i, 0, 0)),
            pl.BlockSpec((BN, h), lambda i: (i, 0)),
            pl.BlockSpec((h, h), lambda i: (0, 0)),
            pl.BlockSpec((h, h), lambda i: (0, 0)),
            pl.BlockSpec((h, h), lambda i: (0, 0)),
            pl.BlockSpec((h, h), lambda i: (0, 0)),
            pl.BlockSpec((h, h), lambda i: (0, 0)),
            pl.BlockSpec((h, h), lambda i: (0, 0)),
        ],
        out_specs=pl.BlockSpec((BN, h), lambda i: (i, 0)),
        out_shape=jax.ShapeDtypeStruct((n, h), jnp.float32),
    )(tid, flat_x, shared_gate_w, shared_up_w, shared_down_w, wga, wua, wda)
    return out.reshape(b, t, h)
